# baseline (speedup 1.0000x reference)
import functools

import jax
import jax.numpy as jnp
from jax import lax
from jax.experimental import pallas as pl
from jax.experimental.pallas import tpu as pltpu
from jax.experimental.pallas import tpu_sc as plsc

V = 100000
D = 64
B = 1024
CTX = 20
MAXN = 1.0

NC, NS, LANES = 2, 16, 16
NW = NC * NS
BPW = B // NW
IPW = BPW * CTX
CHUNK = 128
NCHUNK = IPW // CHUNK
GROUPS = IPW // LANES

_mesh = plsc.VectorSubcoreMesh(
    core_axis_name="c", subcore_axis_name="s", num_cores=NC, num_subcores=NS
)


@functools.partial(
    pl.kernel,
    out_type=jax.ShapeDtypeStruct((B, D), jnp.float32),
    mesh=_mesh,
    scratch_types=[
        pltpu.VMEM((NCHUNK, CHUNK), jnp.int32),
        pltpu.VMEM((IPW, D), jnp.float32),
        pltpu.VMEM((IPW, LANES), jnp.float32),
        pltpu.VMEM((BPW, D), jnp.float32),
        pltpu.SemaphoreType.DMA,
    ],
    compiler_params=pltpu.CompilerParams(
        needs_layout_passes=False, use_tc_tiling_on_sc=False
    ),
)
def _sc_pool(idx_hbm, table_hbm, x_hbm, idx_v, rows_v, scale_b, out_v, sem):
    wid = lax.axis_index("s") * NC + lax.axis_index("c")
    pltpu.sync_copy(idx_hbm.at[wid], idx_v)

    copies = [
        pltpu.async_copy(
            table_hbm.at[idx_v.at[j]], rows_v.at[pl.ds(j * CHUNK, CHUNK)], sem
        )
        for j in range(NCHUNK)
    ]
    for c in copies:
        c.wait()

    def norm_body(g, carry):
        base = g * LANES
        row_ids = base + lax.iota(jnp.int32, LANES)
        acc = jnp.zeros((LANES,), jnp.float32)
        for d in range(D):
            col = jnp.full((LANES,), d, jnp.int32)
            v = plsc.load_gather(rows_v, [row_ids, col])
            acc = acc + v * v
        n2 = jnp.maximum(acc, jnp.float32(1e-24))
        bits = plsc.bitcast(n2, jnp.int32)
        bits = jnp.int32(0x5F3759DF) - lax.shift_right_logical(bits, 1)
        y = plsc.bitcast(bits, jnp.float32)
        for _ in range(3):
            y = y * (jnp.float32(1.5) - jnp.float32(0.5) * n2 * y * y)
        scale = jnp.minimum(jnp.float32(MAXN), y)
        for cc in range(LANES):
            col = jnp.full((LANES,), cc, jnp.int32)
            plsc.store_scatter(scale_b, [row_ids, col], scale)
        return carry

    lax.fori_loop(0, GROUPS, norm_body, 0)

    def mean_body(bi, carry):
        r0 = bi * CTX
        acc = [jnp.zeros((LANES,), jnp.float32) for _ in range(D // LANES)]
        for c in range(CTX):
            sv = scale_b[r0 + c, :]
            for k in range(D // LANES):
                acc[k] = acc[k] + rows_v[r0 + c, pl.ds(k * LANES, LANES)] * sv
        for k in range(D // LANES):
            out_v[bi, pl.ds(k * LANES, LANES)] = acc[k] * jnp.float32(1.0 / CTX)
        return carry

    lax.fori_loop(0, BPW, mean_body, 0)
    pltpu.sync_copy(out_v, x_hbm.at[pl.ds(wid * BPW, BPW)])


BV = 2048
GRID_V = (V + BV - 1) // BV


def _mm_body(x_ref, w_ref, b_ref, o_ref):
    o_ref[...] = (
        lax.dot_general(
            x_ref[...],
            w_ref[...],
            dimension_numbers=(((1,), (1,)), ((), ())),
            preferred_element_type=jnp.float32,
        )
        + b_ref[...]
    )


def _project(x, W, b2):
    return pl.pallas_call(
        _mm_body,
        grid=(GRID_V,),
        in_specs=[
            pl.BlockSpec((B, D), lambda v: (0, 0)),
            pl.BlockSpec((BV, D), lambda v: (v, 0)),
            pl.BlockSpec((1, BV), lambda v: (0, v)),
        ],
        out_specs=pl.BlockSpec((B, BV), lambda v: (0, v)),
        out_shape=jax.ShapeDtypeStruct((B, V), jnp.float32),
    )(x, W, b2)


def kernel(inputs_, emb_table, W, b):
    idx = inputs_.astype(jnp.int32).reshape(NW, NCHUNK, CHUNK)
    x = _sc_pool(idx, emb_table)
    return _project(x, W, b.reshape(1, V))


# speedup vs baseline: 2.5597x; 2.5597x over previous
import functools

import jax
import jax.numpy as jnp
from jax import lax
from jax.experimental import pallas as pl
from jax.experimental.pallas import tpu as pltpu
from jax.experimental.pallas import tpu_sc as plsc

V = 100000
D = 64
B = 1024
CTX = 20
MAXN = 1.0

NC, NS, LANES = 2, 16, 16
NW = NC * NS
BPW = B // NW
IPW = BPW * CTX
CHUNK = 128
NCHUNK = IPW // CHUNK
GROUPS = IPW // LANES

_mesh = plsc.VectorSubcoreMesh(
    core_axis_name="c", subcore_axis_name="s", num_cores=NC, num_subcores=NS
)


@functools.partial(
    pl.kernel,
    out_type=jax.ShapeDtypeStruct((B, D), jnp.float32),
    mesh=_mesh,
    scratch_types=[
        pltpu.VMEM((NCHUNK, CHUNK), jnp.int32),
        pltpu.VMEM((IPW, D), jnp.float32),
        pltpu.VMEM((IPW, LANES), jnp.float32),
        pltpu.VMEM((BPW, D), jnp.float32),
        pltpu.SemaphoreType.DMA,
    ],
    compiler_params=pltpu.CompilerParams(
        needs_layout_passes=False, use_tc_tiling_on_sc=False
    ),
)
def _sc_pool(idx_hbm, table_hbm, x_hbm, idx_v, rows_v, scale_b, out_v, sem):
    wid = lax.axis_index("s") * NC + lax.axis_index("c")
    pltpu.sync_copy(idx_hbm.at[wid], idx_v)

    copies = [
        pltpu.async_copy(
            table_hbm.at[idx_v.at[j]], rows_v.at[pl.ds(j * CHUNK, CHUNK)], sem
        )
        for j in range(NCHUNK)
    ]
    for c in copies:
        c.wait()

    def norm_body(g, carry):
        base = g * LANES
        row_ids = base + lax.iota(jnp.int32, LANES)
        acc = jnp.zeros((LANES,), jnp.float32)
        for d in range(D):
            col = jnp.full((LANES,), d, jnp.int32)
            v = plsc.load_gather(rows_v, [row_ids, col])
            acc = acc + v * v
        n2 = jnp.maximum(acc, jnp.float32(1e-24))
        bits = plsc.bitcast(n2, jnp.int32)
        bits = jnp.int32(0x5F3759DF) - lax.shift_right_logical(bits, 1)
        y = plsc.bitcast(bits, jnp.float32)
        for _ in range(3):
            y = y * (jnp.float32(1.5) - jnp.float32(0.5) * n2 * y * y)
        scale = jnp.minimum(jnp.float32(MAXN), y)
        for cc in range(LANES):
            col = jnp.full((LANES,), cc, jnp.int32)
            plsc.store_scatter(scale_b, [row_ids, col], scale)
        return carry

    lax.fori_loop(0, GROUPS, norm_body, 0)

    def mean_body(bi, carry):
        r0 = bi * CTX
        acc = [jnp.zeros((LANES,), jnp.float32) for _ in range(D // LANES)]
        for c in range(CTX):
            sv = scale_b[r0 + c, :]
            for k in range(D // LANES):
                acc[k] = acc[k] + rows_v[r0 + c, pl.ds(k * LANES, LANES)] * sv
        for k in range(D // LANES):
            out_v[bi, pl.ds(k * LANES, LANES)] = acc[k] * jnp.float32(1.0 / CTX)
        return carry

    lax.fori_loop(0, BPW, mean_body, 0)
    pltpu.sync_copy(out_v, x_hbm.at[pl.ds(wid * BPW, BPW)])


BV = 2048
GRID_V = (V + BV - 1) // BV


def _mm_body(x_ref, wt_ref, b_ref, o_ref):
    o_ref[...] = (
        lax.dot_general(
            wt_ref[...],
            x_ref[...],
            dimension_numbers=(((0,), (1,)), ((), ())),
            preferred_element_type=jnp.float32,
        )
        + jnp.transpose(b_ref[...])
    )


def _project_t(x, Wt, b2):
    return pl.pallas_call(
        _mm_body,
        grid=(GRID_V,),
        in_specs=[
            pl.BlockSpec((B, D), lambda v: (0, 0)),
            pl.BlockSpec((D, BV), lambda v: (0, v)),
            pl.BlockSpec((1, BV), lambda v: (0, v)),
        ],
        out_specs=pl.BlockSpec((BV, B), lambda v: (v, 0)),
        out_shape=jax.ShapeDtypeStruct((V, B), jnp.float32),
    )(x, Wt, b2)


def kernel(inputs_, emb_table, W, b):
    idx = inputs_.astype(jnp.int32).reshape(NW, NCHUNK, CHUNK)
    x = _sc_pool(idx, emb_table)
    out_t = _project_t(x, W.T, b.reshape(1, V))
    return out_t.T


# speedup vs baseline: 2.7157x; 1.0610x over previous
import functools

import jax
import jax.numpy as jnp
from jax import lax
from jax.experimental import pallas as pl
from jax.experimental.pallas import tpu as pltpu
from jax.experimental.pallas import tpu_sc as plsc

V = 100000
D = 64
B = 1024
CTX = 20
MAXN = 1.0

NC, NS, LANES = 2, 16, 16
NW = NC * NS
BPW = B // NW
IPW = BPW * CTX
CHUNK = 128
NCHUNK = IPW // CHUNK
GROUPS = IPW // LANES

_mesh = plsc.VectorSubcoreMesh(
    core_axis_name="c", subcore_axis_name="s", num_cores=NC, num_subcores=NS
)


DP = 128


@functools.partial(
    pl.kernel,
    out_type=jax.ShapeDtypeStruct((B, D), jnp.float32),
    mesh=_mesh,
    scratch_types=[
        pltpu.VMEM((NCHUNK, CHUNK), jnp.int32),
        pltpu.VMEM((IPW, DP), jnp.float32),
        pltpu.VMEM((IPW, LANES), jnp.float32),
        pltpu.VMEM((BPW, D), jnp.float32),
        pltpu.SemaphoreType.DMA,
    ],
    compiler_params=pltpu.CompilerParams(
        needs_layout_passes=False, use_tc_tiling_on_sc=False
    ),
)
def _sc_pool(idx_hbm, table_hbm, x_hbm, idx_v, rows_v, scale_b, out_v, sem):
    wid = lax.axis_index("s") * NC + lax.axis_index("c")
    pltpu.sync_copy(idx_hbm.at[wid], idx_v)

    copies = [
        pltpu.async_copy(
            table_hbm.at[idx_v.at[j]], rows_v.at[pl.ds(j * CHUNK, CHUNK)], sem
        )
        for j in range(NCHUNK)
    ]
    for c in copies:
        c.wait()

    def norm_body(g, carry):
        base = g * LANES
        row_ids = base + lax.iota(jnp.int32, LANES)
        acc = jnp.zeros((LANES,), jnp.float32)
        for d in range(D):
            col = jnp.full((LANES,), d, jnp.int32)
            v = plsc.load_gather(rows_v, [row_ids, col])
            acc = acc + v * v
        n2 = jnp.maximum(acc, jnp.float32(1e-24))
        bits = plsc.bitcast(n2, jnp.int32)
        bits = jnp.int32(0x5F3759DF) - lax.shift_right_logical(bits, 1)
        y = plsc.bitcast(bits, jnp.float32)
        for _ in range(3):
            y = y * (jnp.float32(1.5) - jnp.float32(0.5) * n2 * y * y)
        scale = jnp.minimum(jnp.float32(MAXN), y)
        for cc in range(LANES):
            col = jnp.full((LANES,), cc, jnp.int32)
            plsc.store_scatter(scale_b, [row_ids, col], scale)
        return carry

    lax.fori_loop(0, GROUPS, norm_body, 0)

    def mean_body(bi, carry):
        r0 = bi * CTX
        acc = [jnp.zeros((LANES,), jnp.float32) for _ in range(D // LANES)]
        for c in range(CTX):
            sv = scale_b[r0 + c, :]
            for k in range(D // LANES):
                acc[k] = acc[k] + rows_v[r0 + c, pl.ds(k * LANES, LANES)] * sv
        for k in range(D // LANES):
            out_v[bi, pl.ds(k * LANES, LANES)] = acc[k] * jnp.float32(1.0 / CTX)
        return carry

    lax.fori_loop(0, BPW, mean_body, 0)
    pltpu.sync_copy(out_v, x_hbm.at[pl.ds(wid * BPW, BPW)])


BV = 2048
GRID_V = (V + BV - 1) // BV


def _tr_body(tt_ref, o_ref):
    t = jnp.transpose(tt_ref[...])
    o_ref[...] = jnp.concatenate([t, jnp.zeros((BV, DP - D), jnp.float32)], axis=1)


def _repack_rows(tt):
    return pl.pallas_call(
        _tr_body,
        grid=(GRID_V,),
        in_specs=[pl.BlockSpec((D, BV), lambda v: (0, v))],
        out_specs=pl.BlockSpec((BV, DP), lambda v: (v, 0)),
        out_shape=jax.ShapeDtypeStruct((V, DP), jnp.float32),
    )(tt)


def _mm_body(x_ref, wt_ref, b_ref, o_ref):
    o_ref[...] = (
        lax.dot_general(
            wt_ref[...],
            x_ref[...],
            dimension_numbers=(((0,), (1,)), ((), ())),
            preferred_element_type=jnp.float32,
        )
        + jnp.transpose(b_ref[0])
    )


def _project_t(x, Wt, b_rows):
    return pl.pallas_call(
        _mm_body,
        grid=(GRID_V,),
        in_specs=[
            pl.BlockSpec((B, D), lambda v: (0, 0)),
            pl.BlockSpec((D, BV), lambda v: (0, v)),
            pl.BlockSpec((1, 1, BV), lambda v: (v, 0, 0)),
        ],
        out_specs=pl.BlockSpec((BV, B), lambda v: (v, 0)),
        out_shape=jax.ShapeDtypeStruct((V, B), jnp.float32),
    )(x, Wt, b_rows)


def kernel(inputs_, emb_table, W, b):
    idx = inputs_.astype(jnp.int32).reshape(NW, NCHUNK, CHUNK)
    table128 = _repack_rows(emb_table.T)
    x = _sc_pool(idx, table128)
    b_rows = jnp.pad(b, (0, GRID_V * BV - V)).reshape(GRID_V, 1, BV)
    out_t = _project_t(x, W.T, b_rows)
    return out_t.T


# speedup vs baseline: 2.9736x; 1.0950x over previous
import functools

import jax
import jax.numpy as jnp
from jax import lax
from jax.experimental import pallas as pl
from jax.experimental.pallas import tpu as pltpu
from jax.experimental.pallas import tpu_sc as plsc

V = 100000
D = 64
B = 1024
CTX = 20
MAXN = 1.0

NC, NS, LANES = 2, 16, 16
NW = NC * NS
BPW = B // NW
IPW = BPW * CTX
CHUNK = 128
NCHUNK = IPW // CHUNK
GROUPS = IPW // LANES

_mesh = plsc.VectorSubcoreMesh(
    core_axis_name="c", subcore_axis_name="s", num_cores=NC, num_subcores=NS
)


DP = 128


@functools.partial(
    pl.kernel,
    out_type=jax.ShapeDtypeStruct((B, D), jnp.float32),
    mesh=_mesh,
    scratch_types=[
        pltpu.VMEM((NCHUNK, CHUNK), jnp.int32),
        pltpu.VMEM((IPW, DP), jnp.float32),
        pltpu.VMEM((IPW, LANES), jnp.float32),
        pltpu.VMEM((BPW, D), jnp.float32),
        pltpu.SemaphoreType.DMA,
    ],
    compiler_params=pltpu.CompilerParams(
        needs_layout_passes=False, use_tc_tiling_on_sc=False
    ),
)
def _sc_pool(idx_hbm, table_hbm, x_hbm, idx_v, rows_v, scale_b, out_v, sem):
    wid = lax.axis_index("s") * NC + lax.axis_index("c")
    pltpu.sync_copy(idx_hbm.at[wid], idx_v)

    copies = [
        pltpu.async_copy(
            table_hbm.at[idx_v.at[j]], rows_v.at[pl.ds(j * CHUNK, CHUNK)], sem
        )
        for j in range(NCHUNK)
    ]
    for c in copies:
        c.wait()

    def norm_body(g, carry):
        base = g * LANES
        row_ids = base + lax.iota(jnp.int32, LANES)
        acc = jnp.zeros((LANES,), jnp.float32)
        for d in range(D):
            col = jnp.full((LANES,), d, jnp.int32)
            v = plsc.load_gather(rows_v, [row_ids, col])
            acc = acc + v * v
        n2 = jnp.maximum(acc, jnp.float32(1e-24))
        bits = plsc.bitcast(n2, jnp.int32)
        bits = jnp.int32(0x5F3759DF) - lax.shift_right_logical(bits, 1)
        y = plsc.bitcast(bits, jnp.float32)
        for _ in range(3):
            y = y * (jnp.float32(1.5) - jnp.float32(0.5) * n2 * y * y)
        scale = jnp.minimum(jnp.float32(MAXN), y)
        for cc in range(LANES):
            col = jnp.full((LANES,), cc, jnp.int32)
            plsc.store_scatter(scale_b, [row_ids, col], scale)
        return carry

    lax.fori_loop(0, GROUPS, norm_body, 0)

    def mean_body(bi, carry):
        r0 = bi * CTX
        acc = [jnp.zeros((LANES,), jnp.float32) for _ in range(D // LANES)]
        for c in range(CTX):
            sv = scale_b[r0 + c, :]
            for k in range(D // LANES):
                acc[k] = acc[k] + rows_v[r0 + c, pl.ds(k * LANES, LANES)] * sv
        for k in range(D // LANES):
            out_v[bi, pl.ds(k * LANES, LANES)] = acc[k] * jnp.float32(1.0 / CTX)
        return carry

    lax.fori_loop(0, BPW, mean_body, 0)
    pltpu.sync_copy(out_v, x_hbm.at[pl.ds(wid * BPW, BPW)])


BV = 4096
GRID_V = (V + BV - 1) // BV


BR = 8192
GRID_R = (V + BR - 1) // BR


def _tr_body(tt_ref, o_ref):
    t = jnp.transpose(tt_ref[...])
    o_ref[...] = jnp.concatenate([t, jnp.zeros((BR, DP - D), jnp.float32)], axis=1)


def _repack_rows(tt):
    return pl.pallas_call(
        _tr_body,
        grid=(GRID_R,),
        in_specs=[pl.BlockSpec((D, BR), lambda v: (0, v))],
        out_specs=pl.BlockSpec((BR, DP), lambda v: (v, 0)),
        out_shape=jax.ShapeDtypeStruct((V, DP), jnp.float32),
    )(tt)


def _mm_body(x_ref, wt_ref, b_ref, o_ref):
    o_ref[...] = (
        lax.dot_general(
            wt_ref[...],
            x_ref[...],
            dimension_numbers=(((0,), (1,)), ((), ())),
            preferred_element_type=jnp.float32,
        )
        + jnp.transpose(b_ref[0])
    )


def _project_t(x, Wt, b_rows):
    return pl.pallas_call(
        _mm_body,
        grid=(GRID_V,),
        in_specs=[
            pl.BlockSpec((B, D), lambda v: (0, 0)),
            pl.BlockSpec((D, BV), lambda v: (0, v)),
            pl.BlockSpec((1, 1, BV), lambda v: (v, 0, 0)),
        ],
        out_specs=pl.BlockSpec((BV, B), lambda v: (v, 0)),
        out_shape=jax.ShapeDtypeStruct((V, B), jnp.float32),
    )(x, Wt, b_rows)


def kernel(inputs_, emb_table, W, b):
    idx = inputs_.astype(jnp.int32).reshape(NW, NCHUNK, CHUNK)
    table128 = _repack_rows(emb_table.T)
    x = _sc_pool(idx, table128)
    b_rows = jnp.pad(b, (0, GRID_V * BV - V)).reshape(GRID_V, 1, BV)
    out_t = _project_t(x, W.T, b_rows)
    return out_t.T
